# baseline (speedup 1.0000x reference)
import jax
import jax.numpy as jnp
from jax import lax
from jax.experimental import pallas as pl
from jax.experimental.pallas import tpu as pltpu
from jax.experimental.pallas import tpu_sc as plsc

NB = 10000
EB = 320000
NH1 = 8
D1 = 64
NC2 = 40
D2 = 48

CHUNK = 512
SUB = 128
NSUB = CHUNK // SUB
EBP = 327680
NCHUNKS = EBP // CHUNK
NWORK = 32
N_I = NCHUNKS // NWORK
NBP = 10240
ROWS_PT = NBP // 16


def _leaky(x):
    return jnp.maximum(x, 0.2 * x)


_GDN = lax.GatherDimensionNumbers(
    offset_dims=(), collapsed_slice_dims=(0,), start_index_map=(0,))


def _vgather(v, idx16):
    return lax.gather(v, idx16[:, None], _GDN, (1,),
                      mode=lax.GatherScatterMode.PROMISE_IN_BOUNDS)


def _tc1_body(x_ref, w1_ref, ams_ref, amd_ref, h_ref, as_ref, ad_ref, g_ref):
    h = jnp.dot(x_ref[...], w1_ref[...], preferred_element_type=jnp.float32)
    h_ref[...] = h
    as_ref[...] = jnp.dot(h, ams_ref[...], preferred_element_type=jnp.float32)
    ad_ref[...] = jnp.dot(h, amd_ref[...], preferred_element_type=jnp.float32)
    g_ref[...] = jnp.max(as_ref[...], axis=0, keepdims=True)


def _tc3_body(p_ref, dp_ref, rm_ref, w2_ref, as2_ref, ad2_ref,
              h2_ref, s2_ref, d2_ref, g2_ref):
    i = pl.program_id(0)
    dinv = 1.0 / (dp_ref[0] + dp_ref[1] + 1e-16)
    dinv64 = jnp.dot(dinv, rm_ref[...], preferred_element_type=jnp.float32)
    o = (p_ref[0] + p_ref[1]) * dinv64
    hact = jnp.where(o > 0, o, jnp.exp(o) - 1.0)
    h2 = jnp.dot(hact, w2_ref[...], preferred_element_type=jnp.float32)
    h2_ref[...] = h2
    s2 = jnp.dot(h2, as2_ref[...], preferred_element_type=jnp.float32)
    d2 = jnp.dot(h2, ad2_ref[...], preferred_element_type=jnp.float32)
    s2_ref[...] = s2
    d2_ref[...] = d2
    m = jnp.max(s2, axis=0, keepdims=True)

    @pl.when(i == 0)
    def _():
        g2_ref[...] = m

    @pl.when(i > 0)
    def _():
        g2_ref[...] = jnp.maximum(g2_ref[...], m)


def _tc5_body(p_ref, dp_ref, rm_ref, out_ref):
    dinv = 1.0 / (dp_ref[0] + dp_ref[1] + 1e-16)
    dinv48 = jnp.dot(dinv, rm_ref[...], preferred_element_type=jnp.float32)
    o = (p_ref[0] + p_ref[1]) * dinv48
    t = o[:, :NC2]
    m = jnp.max(t, axis=1, keepdims=True)
    lse = jnp.log(jnp.sum(jnp.exp(t - m), axis=1, keepdims=True))
    out_ref[...] = t - m - lse


def _wid_and_niter():
    cid = lax.axis_index("c")
    sid = lax.axis_index("s")
    wid = sid * 2 + cid
    return cid, sid, wid, N_I


def _zero_shared_slice(zbuf, shared, sid):
    pltpu.sync_copy(zbuf.at[pl.ds(0, CHUNK)],
                    shared.at[pl.ds(sid * ROWS_PT, CHUNK)])
    pltpu.sync_copy(zbuf.at[pl.ds(0, ROWS_PT - CHUNK)],
                    shared.at[pl.ds(sid * ROWS_PT + CHUNK, ROWS_PT - CHUNK)])


def _idx16(ref, j):
    return ref[j // (SUB // 16), pl.ds(16 * (j % (SUB // 16)), 16)]


def _sc_a1_body(src_hbm, dst_hbm, as_hbm, ad_hbm, g_hbm,
                eexp_hbm, dpart_hbm,
                srcv, dstv, asv, adv, eev, gv, dsh, sem):
    cid, sid, wid, n_i = _wid_and_niter()
    zero = jnp.zeros((16,), jnp.float32)

    def zb(j, _):
        eev[j] = zero
        return 0
    lax.fori_loop(0, CHUNK, zb, 0)
    _zero_shared_slice(eev, dsh, sid)
    pltpu.sync_copy(g_hbm, gv)
    plsc.subcore_barrier()

    g = gv[...]

    def chunk_body(i, _):
        c = wid + NWORK * i
        d1 = pltpu.async_copy(src_hbm.at[c], srcv, sem)
        d2 = pltpu.async_copy(dst_hbm.at[c], dstv, sem)
        d1.wait()
        d2.wait()
        ds_ = []
        for k in range(NSUB):
            ds_.append(pltpu.async_copy(
                as_hbm.at[srcv.at[k]], asv.at[pl.ds(SUB * k, SUB)], sem))
            ds_.append(pltpu.async_copy(
                ad_hbm.at[dstv.at[k]], adv.at[pl.ds(SUB * k, SUB)], sem))
        for d in ds_:
            d.wait()

        def inner(r, _):
            s = asv[r]
            a = adv[r]
            e = _leaky(s + a)
            cb = _leaky(g + a)
            eev[r] = jnp.exp(e - cb)
            return 0
        lax.fori_loop(0, CHUNK, inner, 0)

        for k in range(NSUB):
            pltpu.sync_copy(eev.at[pl.ds(SUB * k, SUB)], dsh.at[dstv.at[k]],
                            add=True)
        pltpu.sync_copy(eev, eexp_hbm.at[pl.ds(CHUNK * c, CHUNK)])
        return 0
    lax.fori_loop(0, n_i, chunk_body, 0)

    plsc.subcore_barrier()
    pltpu.sync_copy(dsh.at[pl.ds(sid * ROWS_PT, ROWS_PT)],
                    dpart_hbm.at[cid, pl.ds(sid * ROWS_PT, ROWS_PT)])


def _sc_b1_body(src_hbm, dst_hbm, h_hbm, eexp_hbm,
                opart_hbm,
                srcv0, dstv0, hrow0, eev0,
                srcv1, dstv1, hrow1, eev1,
                osh, sem0, sem1):
    cid, sid, wid, n_i = _wid_and_niter()
    srcvs = (srcv0, srcv1)
    dstvs = (dstv0, dstv1)
    hrs = (hrow0, hrow1)
    eevs = (eev0, eev1)
    sems = (sem0, sem1)
    zero = jnp.zeros((16,), jnp.float32)

    def zb(j, _):
        for t in range(4):
            hrow0[j, pl.ds(16 * t, 16)] = zero
        return 0
    lax.fori_loop(0, CHUNK, zb, 0)
    _zero_shared_slice(hrow0, osh, sid)
    plsc.subcore_barrier()

    q = lax.iota(jnp.int32, 16)
    bidx = [2 * t + lax.shift_right_logical(q, 3) for t in range(4)]

    def fire(b, t):
        c = wid + NWORK * t
        i1 = pltpu.async_copy(src_hbm.at[c], srcvs[b], sems[b])
        i2 = pltpu.async_copy(dst_hbm.at[c], dstvs[b], sems[b])
        i1.wait()
        i2.wait()
        pltpu.async_copy(eexp_hbm.at[pl.ds(CHUNK * c, CHUNK)], eevs[b], sems[b])
        for k in range(NSUB):
            pltpu.async_copy(h_hbm.at[srcvs[b].at[k]],
                             hrs[b].at[pl.ds(SUB * k, SUB)], sems[b])

    def drain(b):
        pltpu.make_async_copy(eexp_hbm.at[pl.ds(0, CHUNK)], eevs[b],
                              sems[b]).wait()
        for k in range(NSUB):
            pltpu.make_async_copy(h_hbm.at[pl.ds(0, SUB)],
                                  hrs[b].at[pl.ds(SUB * k, SUB)],
                                  sems[b]).wait()

    fire(0, 0)

    def pair_body(ip, _):
        for b in (0, 1):
            t = 2 * ip + b
            drain(b)

            @pl.when(t + 1 < N_I)
            def _():
                fire(1 - b, t + 1)

            def inner(r, _):
                al = eevs[b][r]
                for tt in range(4):
                    av = _vgather(al, bidx[tt])
                    hrs[b][r, pl.ds(16 * tt, 16)] = (
                        hrs[b][r, pl.ds(16 * tt, 16)] * av)
                return 0
            lax.fori_loop(0, CHUNK, inner, 0)

            for k in range(NSUB):
                pltpu.sync_copy(hrs[b].at[pl.ds(SUB * k, SUB)],
                                osh.at[dstvs[b].at[k]], add=True)
        return 0
    lax.fori_loop(0, N_I // 2, pair_body, 0)

    plsc.subcore_barrier()
    pltpu.sync_copy(osh.at[pl.ds(sid * ROWS_PT, ROWS_PT)],
                    opart_hbm.at[cid, pl.ds(sid * ROWS_PT, ROWS_PT)])


def _sc_a2_body(src_hbm, dst_hbm, as2_hbm, ad2_hbm, g2_hbm,
                eexp2_hbm, d2part_hbm,
                srcv, dstv, as2t, ad2t, eevc, eevw, gv, dsh2, sem):
    cid, sid, wid, n_i = _wid_and_niter()
    zero = jnp.zeros((16,), jnp.float32)

    def zb(j, _):
        eevw[j] = zero
        return 0
    lax.fori_loop(0, CHUNK, zb, 0)
    _zero_shared_slice(eevw, dsh2, sid)
    pltpu.sync_copy(as2_hbm, as2t)
    pltpu.sync_copy(ad2_hbm, ad2t)
    pltpu.sync_copy(g2_hbm, gv)
    plsc.subcore_barrier()

    g = gv[...]
    q = lax.iota(jnp.int32, 16)
    zcol = q * 0

    def chunk_body(i, _):
        c = wid + NWORK * i
        d1 = pltpu.async_copy(src_hbm.at[c], srcv, sem)
        d2 = pltpu.async_copy(dst_hbm.at[c], dstv, sem)
        d1.wait()
        d2.wait()

        def inner(j, _):
            s16 = _idx16(srcv, j)
            d16 = _idx16(dstv, j)
            a_s = plsc.load_gather(as2t, [s16])
            a_d = plsc.load_gather(ad2t, [d16])
            e = _leaky(a_s + a_d)
            cb = _leaky(g + a_d)
            ee = jnp.exp(e - cb)
            eevc[j] = ee
            plsc.store_scatter(eevw, [16 * j + q, zcol], ee)
            return 0
        lax.fori_loop(0, CHUNK // 16, inner, 0)

        for k in range(NSUB):
            pltpu.sync_copy(eevw.at[pl.ds(SUB * k, SUB)], dsh2.at[dstv.at[k]],
                            add=True)
        pltpu.sync_copy(eevc, eexp2_hbm.at[pl.ds(CHUNK // 16 * c, CHUNK // 16)])
        return 0
    lax.fori_loop(0, n_i, chunk_body, 0)

    plsc.subcore_barrier()
    pltpu.sync_copy(dsh2.at[pl.ds(sid * ROWS_PT, ROWS_PT)],
                    d2part_hbm.at[cid, pl.ds(sid * ROWS_PT, ROWS_PT)])


def _sc_b2_body(src_hbm, dst_hbm, h2_hbm, eexp2_hbm,
                o2part_hbm,
                srcv, dstv, hrows, eevc, osh2, sem):
    cid, sid, wid, n_i = _wid_and_niter()
    zero = jnp.zeros((16,), jnp.float32)

    def zb(j, _):
        for t in range(3):
            hrows[j, pl.ds(16 * t, 16)] = zero
        return 0
    lax.fori_loop(0, CHUNK, zb, 0)
    _zero_shared_slice(hrows, osh2, sid)
    plsc.subcore_barrier()

    q = lax.iota(jnp.int32, 16)
    sidx = [q * 0 + k for k in range(16)]

    def chunk_body(i, _):
        c = wid + NWORK * i
        d1 = pltpu.async_copy(src_hbm.at[c], srcv, sem)
        d2 = pltpu.async_copy(dst_hbm.at[c], dstv, sem)
        d1.wait()
        d2.wait()
        ds_ = [pltpu.async_copy(
            eexp2_hbm.at[pl.ds(CHUNK // 16 * c, CHUNK // 16)], eevc, sem)]
        for k in range(NSUB):
            ds_.append(pltpu.async_copy(
                h2_hbm.at[srcv.at[k]], hrows.at[pl.ds(SUB * k, SUB)], sem))
        for d in ds_:
            d.wait()

        def inner(j, _):
            al = eevc[j]
            for k in range(16):
                av = _vgather(al, sidx[k])
                r = 16 * j + k
                for t in range(3):
                    hrows[r, pl.ds(16 * t, 16)] = (
                        hrows[r, pl.ds(16 * t, 16)] * av)
            return 0
        lax.fori_loop(0, CHUNK // 16, inner, 0)

        for k in range(NSUB):
            pltpu.sync_copy(hrows.at[pl.ds(SUB * k, SUB)], osh2.at[dstv.at[k]],
                            add=True)
        return 0
    lax.fori_loop(0, n_i, chunk_body, 0)

    plsc.subcore_barrier()
    pltpu.sync_copy(osh2.at[pl.ds(sid * ROWS_PT, ROWS_PT)],
                    o2part_hbm.at[cid, pl.ds(sid * ROWS_PT, ROWS_PT)])


def kernel(x, edge_index, W1, a1_src, a1_dst, W2, a2_src, a2_dst):
    f32 = jnp.float32
    i32 = jnp.int32
    src = edge_index[0].astype(i32)
    dst = edge_index[1].astype(i32)
    pad_e = jnp.full((EBP - EB,), NB, i32)
    src2 = jnp.concatenate([src, pad_e]).reshape(NCHUNKS, NSUB, SUB)
    dst2 = jnp.concatenate([dst, pad_e]).reshape(NCHUNKS, NSUB, SUB)

    eye = jnp.eye(NH1, dtype=f32)
    ams = jnp.pad((a1_src[:, :, None] * eye[:, None, :]).reshape(D1, NH1),
                  ((0, 0), (0, 8)))
    amd = jnp.pad((a1_dst[:, :, None] * eye[:, None, :]).reshape(D1, NH1),
                  ((0, 0), (0, 8)))

    h1, as16, ad16, g8 = pl.pallas_call(
        _tc1_body,
        out_shape=(
            jax.ShapeDtypeStruct((NB, D1), f32),
            jax.ShapeDtypeStruct((NB, 16), f32),
            jax.ShapeDtypeStruct((NB, 16), f32),
            jax.ShapeDtypeStruct((1, 16), f32),
        ),
    )(x, W1, ams, amd)
    g16 = jnp.where(jnp.arange(16) < NH1, g8[0], 40.0)
    as16p = jnp.pad(as16, ((0, NBP - NB), (0, 0)))
    ad16p = jnp.pad(ad16, ((0, NBP - NB), (0, 0)))
    h1p = jnp.pad(h1, ((0, NBP - NB), (0, 0)))

    mesh = plsc.VectorSubcoreMesh(core_axis_name="c", subcore_axis_name="s")
    idx_t = pltpu.VMEM((NSUB, SUB), i32)
    sc_params = pltpu.CompilerParams(use_tc_tiling_on_sc=False,
                                     needs_layout_passes=False)

    eexp1, dpart = pl.kernel(
        _sc_a1_body,
        out_type=(
            jax.ShapeDtypeStruct((EBP, 16), f32),
            jax.ShapeDtypeStruct((2, NBP, 16), f32),
        ),
        mesh=mesh,
        compiler_params=sc_params,
        scratch_types=[
            idx_t, idx_t,
            pltpu.VMEM((CHUNK, 16), f32),
            pltpu.VMEM((CHUNK, 16), f32),
            pltpu.VMEM((CHUNK, 16), f32),
            pltpu.VMEM((16,), f32),
            pltpu.VMEM_SHARED((NBP, 16), f32),
            pltpu.SemaphoreType.DMA,
        ],
    )(src2, dst2, as16p, ad16p, g16)

    opart = pl.kernel(
        _sc_b1_body,
        out_type=jax.ShapeDtypeStruct((2, NBP, D1), f32),
        mesh=mesh,
        compiler_params=sc_params,
        scratch_types=[
            idx_t, idx_t,
            pltpu.VMEM((CHUNK, D1), f32),
            pltpu.VMEM((CHUNK, 16), f32),
            idx_t, idx_t,
            pltpu.VMEM((CHUNK, D1), f32),
            pltpu.VMEM((CHUNK, 16), f32),
            pltpu.VMEM_SHARED((NBP, D1), f32),
            pltpu.SemaphoreType.DMA,
            pltpu.SemaphoreType.DMA,
        ],
    )(src2, dst2, h1p, eexp1)

    w2p = jnp.pad(W2, ((0, 0), (0, D2 - NC2)))
    as2v = jnp.pad(a2_src[0], (0, D2 - NC2)).reshape(D2, 1)
    ad2v = jnp.pad(a2_dst[0], (0, D2 - NC2)).reshape(D2, 1)

    rm1 = (jnp.arange(D1)[None, :] // 8 ==
           jnp.arange(16)[:, None]).astype(f32)
    b3 = 1024
    h2, s2, d2, g2 = pl.pallas_call(
        _tc3_body,
        grid=(NBP // b3,),
        in_specs=[pl.BlockSpec((2, b3, D1), lambda i: (0, i, 0)),
                  pl.BlockSpec((2, b3, 16), lambda i: (0, i, 0)),
                  pl.BlockSpec((16, D1), lambda i: (0, 0)),
                  pl.BlockSpec((D1, D2), lambda i: (0, 0)),
                  pl.BlockSpec((D2, 1), lambda i: (0, 0)),
                  pl.BlockSpec((D2, 1), lambda i: (0, 0))],
        out_specs=(
            pl.BlockSpec((b3, D2), lambda i: (i, 0)),
            pl.BlockSpec((b3, 1), lambda i: (i, 0)),
            pl.BlockSpec((b3, 1), lambda i: (i, 0)),
            pl.BlockSpec((1, 1), lambda i: (0, 0)),
        ),
        out_shape=(
            jax.ShapeDtypeStruct((NBP, D2), f32),
            jax.ShapeDtypeStruct((NBP, 1), f32),
            jax.ShapeDtypeStruct((NBP, 1), f32),
            jax.ShapeDtypeStruct((1, 1), f32),
        ),
    )(opart, dpart, rm1, w2p, as2v, ad2v)
    g2_16 = jnp.broadcast_to(g2[0, 0], (16,))

    eexp2, d2part = pl.kernel(
        _sc_a2_body,
        out_type=(
            jax.ShapeDtypeStruct((EBP // 16, 16), f32),
            jax.ShapeDtypeStruct((2, NBP, 16), f32),
        ),
        mesh=mesh,
        compiler_params=sc_params,
        scratch_types=[
            idx_t, idx_t,
            pltpu.VMEM((NBP,), f32),
            pltpu.VMEM((NBP,), f32),
            pltpu.VMEM((CHUNK // 16, 16), f32),
            pltpu.VMEM((CHUNK, 16), f32),
            pltpu.VMEM((16,), f32),
            pltpu.VMEM_SHARED((NBP, 16), f32),
            pltpu.SemaphoreType.DMA,
        ],
    )(src2, dst2, s2.reshape(NBP), d2.reshape(NBP), g2_16)

    o2part = pl.kernel(
        _sc_b2_body,
        out_type=jax.ShapeDtypeStruct((2, NBP, D2), f32),
        mesh=mesh,
        compiler_params=sc_params,
        scratch_types=[
            idx_t, idx_t,
            pltpu.VMEM((CHUNK, D2), f32),
            pltpu.VMEM((CHUNK // 16, 16), f32),
            pltpu.VMEM_SHARED((NBP, D2), f32),
            pltpu.SemaphoreType.DMA,
        ],
    )(src2, dst2, h2, eexp2)

    rm2 = (jnp.arange(16)[:, None] * jnp.ones((1, D2)) == 0).astype(f32)
    b5 = 1000
    out = pl.pallas_call(
        _tc5_body,
        grid=(NB // b5,),
        in_specs=[pl.BlockSpec((2, b5, D2), lambda i: (0, i, 0)),
                  pl.BlockSpec((2, b5, 16), lambda i: (0, i, 0)),
                  pl.BlockSpec((16, D2), lambda i: (0, 0))],
        out_specs=pl.BlockSpec((b5, NC2), lambda i: (i, 0)),
        out_shape=jax.ShapeDtypeStruct((NB, NC2), f32),
    )(o2part, d2part, rm2)
    return out


# speedup vs baseline: 1.5415x; 1.5415x over previous
import jax
import jax.numpy as jnp
from jax import lax
from jax.experimental import pallas as pl
from jax.experimental.pallas import tpu as pltpu
from jax.experimental.pallas import tpu_sc as plsc

NB = 10000
EB = 320000
NH1 = 8
D1 = 64
NC2 = 40
D2 = 48

CHUNK = 512
SUB = 128
NSUB = CHUNK // SUB
EBP = 327680
NCHUNKS = EBP // CHUNK
NWORK = 32
N_I = NCHUNKS // NWORK
NBP = 10240
ROWS_PT = NBP // 16


def _leaky(x):
    return jnp.maximum(x, 0.2 * x)


_GDN = lax.GatherDimensionNumbers(
    offset_dims=(), collapsed_slice_dims=(0,), start_index_map=(0,))


def _vgather(v, idx16):
    return lax.gather(v, idx16[:, None], _GDN, (1,),
                      mode=lax.GatherScatterMode.PROMISE_IN_BOUNDS)


def _tc1_body(x_ref, w1_ref, ams_ref, amd_ref, h_ref, as_ref, ad_ref, g_ref):
    h = jnp.dot(x_ref[...], w1_ref[...], preferred_element_type=jnp.float32)
    h_ref[...] = h
    as_ref[...] = jnp.dot(h, ams_ref[...], preferred_element_type=jnp.float32)
    ad_ref[...] = jnp.dot(h, amd_ref[...], preferred_element_type=jnp.float32)
    g_ref[...] = jnp.max(as_ref[...], axis=0, keepdims=True)


def _tc3_body(p_ref, dp_ref, rm_ref, w2_ref, as2_ref, ad2_ref,
              h2_ref, s2_ref, d2_ref, g2_ref):
    i = pl.program_id(0)
    dinv = 1.0 / (dp_ref[0] + dp_ref[1] + 1e-16)
    dinv64 = jnp.dot(dinv, rm_ref[...], preferred_element_type=jnp.float32)
    o = (p_ref[0] + p_ref[1]) * dinv64
    hact = jnp.where(o > 0, o, jnp.exp(o) - 1.0)
    h2 = jnp.dot(hact, w2_ref[...], preferred_element_type=jnp.float32)
    h2_ref[...] = h2
    s2 = jnp.dot(h2, as2_ref[...], preferred_element_type=jnp.float32)
    d2 = jnp.dot(h2, ad2_ref[...], preferred_element_type=jnp.float32)
    s2_ref[...] = s2
    d2_ref[...] = d2
    m = jnp.max(s2, axis=0, keepdims=True)

    @pl.when(i == 0)
    def _():
        g2_ref[...] = m

    @pl.when(i > 0)
    def _():
        g2_ref[...] = jnp.maximum(g2_ref[...], m)


def _tc5_body(p_ref, dp_ref, rm_ref, out_ref):
    dinv = 1.0 / (dp_ref[0] + dp_ref[1] + 1e-16)
    dinv48 = jnp.dot(dinv, rm_ref[...], preferred_element_type=jnp.float32)
    o = (p_ref[0] + p_ref[1]) * dinv48
    t = o[:, :NC2]
    m = jnp.max(t, axis=1, keepdims=True)
    lse = jnp.log(jnp.sum(jnp.exp(t - m), axis=1, keepdims=True))
    out_ref[...] = t - m - lse


def _wid_and_niter():
    cid = lax.axis_index("c")
    sid = lax.axis_index("s")
    wid = sid * 2 + cid
    return cid, sid, wid, N_I


def _zero_shared_slice(zbuf, shared, sid):
    pltpu.sync_copy(zbuf.at[pl.ds(0, CHUNK)],
                    shared.at[pl.ds(sid * ROWS_PT, CHUNK)])
    pltpu.sync_copy(zbuf.at[pl.ds(0, ROWS_PT - CHUNK)],
                    shared.at[pl.ds(sid * ROWS_PT + CHUNK, ROWS_PT - CHUNK)])


def _idx16(ref, j):
    return ref[j // (SUB // 16), pl.ds(16 * (j % (SUB // 16)), 16)]


def _sc_a1_body(src_hbm, dst_hbm, as_hbm, ad_hbm, g_hbm,
                eexp_hbm, dpart_hbm,
                srcv, dstv, asv, adv, eev, gv, dsh, sem):
    cid, sid, wid, n_i = _wid_and_niter()
    zero = jnp.zeros((16,), jnp.float32)

    def zb(j, _):
        eev[j] = zero
        return 0
    lax.fori_loop(0, CHUNK, zb, 0)
    _zero_shared_slice(eev, dsh, sid)
    pltpu.sync_copy(g_hbm, gv)
    plsc.subcore_barrier()

    g = gv[...]

    def chunk_body(i, _):
        c = wid + NWORK * i
        d1 = pltpu.async_copy(src_hbm.at[c], srcv, sem)
        d2 = pltpu.async_copy(dst_hbm.at[c], dstv, sem)
        d1.wait()
        d2.wait()
        ds_ = []
        for k in range(NSUB):
            ds_.append(pltpu.async_copy(
                as_hbm.at[srcv.at[k]], asv.at[pl.ds(SUB * k, SUB)], sem))
            ds_.append(pltpu.async_copy(
                ad_hbm.at[dstv.at[k]], adv.at[pl.ds(SUB * k, SUB)], sem))
        for d in ds_:
            d.wait()

        def inner(r, _):
            s = asv[r]
            a = adv[r]
            e = _leaky(s + a)
            cb = _leaky(g + a)
            eev[r] = jnp.exp(e - cb)
            return 0
        lax.fori_loop(0, CHUNK, inner, 0)

        for k in range(NSUB):
            pltpu.sync_copy(eev.at[pl.ds(SUB * k, SUB)], dsh.at[dstv.at[k]],
                            add=True)
        pltpu.sync_copy(eev, eexp_hbm.at[pl.ds(CHUNK * c, CHUNK)])
        return 0
    lax.fori_loop(0, n_i, chunk_body, 0)

    plsc.subcore_barrier()
    pltpu.sync_copy(dsh.at[pl.ds(sid * ROWS_PT, ROWS_PT)],
                    dpart_hbm.at[cid, pl.ds(sid * ROWS_PT, ROWS_PT)])


def _sc_b1_body(src_hbm, dst_hbm, h_hbm, eexp_hbm,
                opart_hbm,
                srcv0, dstv0, hrow0, eev0,
                srcv1, dstv1, hrow1, eev1,
                osh, sem0, sem1):
    cid, sid, wid, n_i = _wid_and_niter()
    srcvs = (srcv0, srcv1)
    dstvs = (dstv0, dstv1)
    hrs = (hrow0, hrow1)
    eevs = (eev0, eev1)
    sems = (sem0, sem1)
    zero = jnp.zeros((16,), jnp.float32)

    def zb(j, _):
        for t in range(4):
            hrow0[j, pl.ds(16 * t, 16)] = zero
        return 0
    lax.fori_loop(0, CHUNK, zb, 0)
    _zero_shared_slice(hrow0, osh, sid)
    plsc.subcore_barrier()

    q = lax.iota(jnp.int32, 16)
    bidx = [2 * t + lax.shift_right_logical(q, 3) for t in range(4)]

    def fire(b, t):
        c = wid + NWORK * t
        i1 = pltpu.async_copy(src_hbm.at[c], srcvs[b], sems[b])
        i2 = pltpu.async_copy(dst_hbm.at[c], dstvs[b], sems[b])
        i1.wait()
        i2.wait()
        pltpu.async_copy(eexp_hbm.at[pl.ds(CHUNK * c, CHUNK)], eevs[b], sems[b])
        for k in range(NSUB):
            pltpu.async_copy(h_hbm.at[srcvs[b].at[k]],
                             hrs[b].at[pl.ds(SUB * k, SUB)], sems[b])

    def drain(b):
        pltpu.make_async_copy(eexp_hbm.at[pl.ds(0, CHUNK)], eevs[b],
                              sems[b]).wait()
        for k in range(NSUB):
            pltpu.make_async_copy(h_hbm.at[pl.ds(0, SUB)],
                                  hrs[b].at[pl.ds(SUB * k, SUB)],
                                  sems[b]).wait()

    fire(0, 0)

    def pair_body(ip, _):
        for b in (0, 1):
            t = 2 * ip + b
            drain(b)

            @pl.when(t + 1 < N_I)
            def _():
                fire(1 - b, t + 1)

            def inner(r, _):
                al = eevs[b][r]
                for tt in range(4):
                    av = _vgather(al, bidx[tt])
                    hrs[b][r, pl.ds(16 * tt, 16)] = (
                        hrs[b][r, pl.ds(16 * tt, 16)] * av)
                return 0
            lax.fori_loop(0, CHUNK, inner, 0)

            for k in range(NSUB):
                pltpu.sync_copy(hrs[b].at[pl.ds(SUB * k, SUB)],
                                osh.at[dstvs[b].at[k]], add=True)
        return 0
    lax.fori_loop(0, N_I // 2, pair_body, 0)

    plsc.subcore_barrier()
    pltpu.sync_copy(osh.at[pl.ds(sid * ROWS_PT, ROWS_PT)],
                    opart_hbm.at[cid, pl.ds(sid * ROWS_PT, ROWS_PT)])


def _sc_a2_body(src_hbm, dst_hbm, as2_hbm, ad2_hbm, g2_hbm,
                eexp2_hbm, d2part_hbm,
                srcv, dstv, as2t, ad2t, eevc, eevw, gv, dsh2, sem):
    cid, sid, wid, n_i = _wid_and_niter()
    zero = jnp.zeros((16,), jnp.float32)

    def zb(j, _):
        eevw[j] = zero
        return 0
    lax.fori_loop(0, CHUNK, zb, 0)
    _zero_shared_slice(eevw, dsh2, sid)
    pltpu.sync_copy(as2_hbm, as2t)
    pltpu.sync_copy(ad2_hbm, ad2t)
    pltpu.sync_copy(g2_hbm, gv)
    plsc.subcore_barrier()

    g = gv[...]
    q = lax.iota(jnp.int32, 16)
    zcol = q * 0

    def chunk_body(i, _):
        c = wid + NWORK * i
        d1 = pltpu.async_copy(src_hbm.at[c], srcv, sem)
        d2 = pltpu.async_copy(dst_hbm.at[c], dstv, sem)
        d1.wait()
        d2.wait()

        def inner(j, _):
            s16 = _idx16(srcv, j)
            d16 = _idx16(dstv, j)
            a_s = plsc.load_gather(as2t, [s16])
            a_d = plsc.load_gather(ad2t, [d16])
            e = _leaky(a_s + a_d)
            cb = _leaky(g + a_d)
            ee = jnp.exp(e - cb)
            eevc[j] = ee
            plsc.store_scatter(eevw, [16 * j + q, zcol], ee)
            return 0
        lax.fori_loop(0, CHUNK // 16, inner, 0)

        for k in range(NSUB):
            pltpu.sync_copy(eevw.at[pl.ds(SUB * k, SUB)], dsh2.at[dstv.at[k]],
                            add=True)
        pltpu.sync_copy(eevc, eexp2_hbm.at[pl.ds(CHUNK // 16 * c, CHUNK // 16)])
        return 0
    lax.fori_loop(0, n_i, chunk_body, 0)

    plsc.subcore_barrier()
    pltpu.sync_copy(dsh2.at[pl.ds(sid * ROWS_PT, ROWS_PT)],
                    d2part_hbm.at[cid, pl.ds(sid * ROWS_PT, ROWS_PT)])


def _sc_b2_body(src_hbm, dst_hbm, h2_hbm, eexp2_hbm,
                o2part_hbm,
                srcv, dstv, hrows, eevc, osh2, sem):
    cid, sid, wid, n_i = _wid_and_niter()
    zero = jnp.zeros((16,), jnp.float32)

    def zb(j, _):
        for t in range(3):
            hrows[j, pl.ds(16 * t, 16)] = zero
        return 0
    lax.fori_loop(0, CHUNK, zb, 0)
    _zero_shared_slice(hrows, osh2, sid)
    plsc.subcore_barrier()

    q = lax.iota(jnp.int32, 16)
    sidx = [q * 0 + k for k in range(16)]

    def chunk_body(i, _):
        c = wid + NWORK * i
        d1 = pltpu.async_copy(src_hbm.at[c], srcv, sem)
        d2 = pltpu.async_copy(dst_hbm.at[c], dstv, sem)
        d1.wait()
        d2.wait()
        ds_ = [pltpu.async_copy(
            eexp2_hbm.at[pl.ds(CHUNK // 16 * c, CHUNK // 16)], eevc, sem)]
        for k in range(NSUB):
            ds_.append(pltpu.async_copy(
                h2_hbm.at[srcv.at[k]], hrows.at[pl.ds(SUB * k, SUB)], sem))
        for d in ds_:
            d.wait()

        def inner(j, _):
            al = eevc[j]
            for k in range(16):
                av = _vgather(al, sidx[k])
                r = 16 * j + k
                for t in range(3):
                    hrows[r, pl.ds(16 * t, 16)] = (
                        hrows[r, pl.ds(16 * t, 16)] * av)
            return 0
        lax.fori_loop(0, CHUNK // 16, inner, 0)

        for k in range(NSUB):
            pltpu.sync_copy(hrows.at[pl.ds(SUB * k, SUB)], osh2.at[dstv.at[k]],
                            add=True)
        return 0
    lax.fori_loop(0, n_i, chunk_body, 0)

    plsc.subcore_barrier()
    pltpu.sync_copy(osh2.at[pl.ds(sid * ROWS_PT, ROWS_PT)],
                    o2part_hbm.at[cid, pl.ds(sid * ROWS_PT, ROWS_PT)])


def kernel(x, edge_index, W1, a1_src, a1_dst, W2, a2_src, a2_dst):
    f32 = jnp.float32
    i32 = jnp.int32
    src = edge_index[0].astype(i32)
    dst = edge_index[1].astype(i32)
    pad_e = NB + jnp.arange(EBP - EB, dtype=i32) % (NBP - NB)
    src2 = jnp.concatenate([src, pad_e]).reshape(NCHUNKS, NSUB, SUB)
    dst2 = jnp.concatenate([dst, pad_e]).reshape(NCHUNKS, NSUB, SUB)

    eye = jnp.eye(NH1, dtype=f32)
    ams = jnp.pad((a1_src[:, :, None] * eye[:, None, :]).reshape(D1, NH1),
                  ((0, 0), (0, 8)))
    amd = jnp.pad((a1_dst[:, :, None] * eye[:, None, :]).reshape(D1, NH1),
                  ((0, 0), (0, 8)))

    h1, as16, ad16, g8 = pl.pallas_call(
        _tc1_body,
        out_shape=(
            jax.ShapeDtypeStruct((NB, D1), f32),
            jax.ShapeDtypeStruct((NB, 16), f32),
            jax.ShapeDtypeStruct((NB, 16), f32),
            jax.ShapeDtypeStruct((1, 16), f32),
        ),
    )(x, W1, ams, amd)
    g16 = jnp.where(jnp.arange(16) < NH1, g8[0], 40.0)
    as16p = jnp.pad(as16, ((0, NBP - NB), (0, 0)))
    ad16p = jnp.pad(ad16, ((0, NBP - NB), (0, 0)))
    h1p = jnp.pad(h1, ((0, NBP - NB), (0, 0)))

    mesh = plsc.VectorSubcoreMesh(core_axis_name="c", subcore_axis_name="s")
    idx_t = pltpu.VMEM((NSUB, SUB), i32)
    sc_params = pltpu.CompilerParams(use_tc_tiling_on_sc=False,
                                     needs_layout_passes=False)

    eexp1, dpart = pl.kernel(
        _sc_a1_body,
        out_type=(
            jax.ShapeDtypeStruct((EBP, 16), f32),
            jax.ShapeDtypeStruct((2, NBP, 16), f32),
        ),
        mesh=mesh,
        compiler_params=sc_params,
        scratch_types=[
            idx_t, idx_t,
            pltpu.VMEM((CHUNK, 16), f32),
            pltpu.VMEM((CHUNK, 16), f32),
            pltpu.VMEM((CHUNK, 16), f32),
            pltpu.VMEM((16,), f32),
            pltpu.VMEM_SHARED((NBP, 16), f32),
            pltpu.SemaphoreType.DMA,
        ],
    )(src2, dst2, as16p, ad16p, g16)

    opart = pl.kernel(
        _sc_b1_body,
        out_type=jax.ShapeDtypeStruct((2, NBP, D1), f32),
        mesh=mesh,
        compiler_params=sc_params,
        scratch_types=[
            idx_t, idx_t,
            pltpu.VMEM((CHUNK, D1), f32),
            pltpu.VMEM((CHUNK, 16), f32),
            idx_t, idx_t,
            pltpu.VMEM((CHUNK, D1), f32),
            pltpu.VMEM((CHUNK, 16), f32),
            pltpu.VMEM_SHARED((NBP, D1), f32),
            pltpu.SemaphoreType.DMA,
            pltpu.SemaphoreType.DMA,
        ],
    )(src2, dst2, h1p, eexp1)

    w2p = jnp.pad(W2, ((0, 0), (0, D2 - NC2)))
    as2v = jnp.pad(a2_src[0], (0, D2 - NC2)).reshape(D2, 1)
    ad2v = jnp.pad(a2_dst[0], (0, D2 - NC2)).reshape(D2, 1)

    rm1 = (jnp.arange(D1)[None, :] // 8 ==
           jnp.arange(16)[:, None]).astype(f32)
    b3 = 1024
    h2, s2, d2, g2 = pl.pallas_call(
        _tc3_body,
        grid=(NBP // b3,),
        in_specs=[pl.BlockSpec((2, b3, D1), lambda i: (0, i, 0)),
                  pl.BlockSpec((2, b3, 16), lambda i: (0, i, 0)),
                  pl.BlockSpec((16, D1), lambda i: (0, 0)),
                  pl.BlockSpec((D1, D2), lambda i: (0, 0)),
                  pl.BlockSpec((D2, 1), lambda i: (0, 0)),
                  pl.BlockSpec((D2, 1), lambda i: (0, 0))],
        out_specs=(
            pl.BlockSpec((b3, D2), lambda i: (i, 0)),
            pl.BlockSpec((b3, 1), lambda i: (i, 0)),
            pl.BlockSpec((b3, 1), lambda i: (i, 0)),
            pl.BlockSpec((1, 1), lambda i: (0, 0)),
        ),
        out_shape=(
            jax.ShapeDtypeStruct((NBP, D2), f32),
            jax.ShapeDtypeStruct((NBP, 1), f32),
            jax.ShapeDtypeStruct((NBP, 1), f32),
            jax.ShapeDtypeStruct((1, 1), f32),
        ),
    )(opart, dpart, rm1, w2p, as2v, ad2v)
    g2_16 = jnp.broadcast_to(g2[0, 0], (16,))

    eexp2, d2part = pl.kernel(
        _sc_a2_body,
        out_type=(
            jax.ShapeDtypeStruct((EBP // 16, 16), f32),
            jax.ShapeDtypeStruct((2, NBP, 16), f32),
        ),
        mesh=mesh,
        compiler_params=sc_params,
        scratch_types=[
            idx_t, idx_t,
            pltpu.VMEM((NBP,), f32),
            pltpu.VMEM((NBP,), f32),
            pltpu.VMEM((CHUNK // 16, 16), f32),
            pltpu.VMEM((CHUNK, 16), f32),
            pltpu.VMEM((16,), f32),
            pltpu.VMEM_SHARED((NBP, 16), f32),
            pltpu.SemaphoreType.DMA,
        ],
    )(src2, dst2, s2.reshape(NBP), d2.reshape(NBP), g2_16)

    o2part = pl.kernel(
        _sc_b2_body,
        out_type=jax.ShapeDtypeStruct((2, NBP, D2), f32),
        mesh=mesh,
        compiler_params=sc_params,
        scratch_types=[
            idx_t, idx_t,
            pltpu.VMEM((CHUNK, D2), f32),
            pltpu.VMEM((CHUNK // 16, 16), f32),
            pltpu.VMEM_SHARED((NBP, D2), f32),
            pltpu.SemaphoreType.DMA,
        ],
    )(src2, dst2, h2, eexp2)

    rm2 = (jnp.arange(16)[:, None] * jnp.ones((1, D2)) == 0).astype(f32)
    b5 = 1000
    out = pl.pallas_call(
        _tc5_body,
        grid=(NB // b5,),
        in_specs=[pl.BlockSpec((2, b5, D2), lambda i: (0, i, 0)),
                  pl.BlockSpec((2, b5, 16), lambda i: (0, i, 0)),
                  pl.BlockSpec((16, D2), lambda i: (0, 0))],
        out_specs=pl.BlockSpec((b5, NC2), lambda i: (i, 0)),
        out_shape=jax.ShapeDtypeStruct((NB, NC2), f32),
    )(o2part, d2part, rm2)
    return out


# speedup vs baseline: 1.7703x; 1.1484x over previous
import jax
import jax.numpy as jnp
from jax import lax
from jax.experimental import pallas as pl
from jax.experimental.pallas import tpu as pltpu
from jax.experimental.pallas import tpu_sc as plsc

NB = 10000
EB = 320000
NH1 = 8
D1 = 64
NC2 = 40
D2 = 48

CHUNK = 512
SUB = 128
NSUB = CHUNK // SUB
EBP = 327680
NCHUNKS = EBP // CHUNK
NWORK = 32
N_I = NCHUNKS // NWORK
NBP = 10240
ROWS_PT = NBP // 16


def _leaky(x):
    return jnp.maximum(x, 0.2 * x)


_GDN = lax.GatherDimensionNumbers(
    offset_dims=(), collapsed_slice_dims=(0,), start_index_map=(0,))


def _vgather(v, idx16):
    return lax.gather(v, idx16[:, None], _GDN, (1,),
                      mode=lax.GatherScatterMode.PROMISE_IN_BOUNDS)


def _tc1_body(x_ref, w1_ref, ams_ref, amd_ref, h_ref, as_ref, ad_ref, g_ref):
    h = jnp.dot(x_ref[...], w1_ref[...], preferred_element_type=jnp.float32)
    h_ref[...] = h
    as_ref[...] = jnp.dot(h, ams_ref[...], preferred_element_type=jnp.float32)
    ad_ref[...] = jnp.dot(h, amd_ref[...], preferred_element_type=jnp.float32)
    g_ref[...] = jnp.max(as_ref[...], axis=0, keepdims=True)


def _tc3_body(p_ref, dp_ref, rm_ref, w2_ref, as2_ref, ad2_ref,
              h2_ref, s2_ref, d2_ref, g2_ref):
    i = pl.program_id(0)
    dinv = 1.0 / (dp_ref[0] + dp_ref[1] + 1e-16)
    dinv64 = jnp.dot(dinv, rm_ref[...], preferred_element_type=jnp.float32)
    o = (p_ref[0] + p_ref[1]) * dinv64
    hact = jnp.where(o > 0, o, jnp.exp(o) - 1.0)
    h2 = jnp.dot(hact, w2_ref[...], preferred_element_type=jnp.float32)
    h2_ref[...] = h2
    s2 = jnp.dot(h2, as2_ref[...], preferred_element_type=jnp.float32)
    d2 = jnp.dot(h2, ad2_ref[...], preferred_element_type=jnp.float32)
    s2_ref[...] = s2
    d2_ref[...] = d2
    m = jnp.max(s2, axis=0, keepdims=True)

    @pl.when(i == 0)
    def _():
        g2_ref[...] = m

    @pl.when(i > 0)
    def _():
        g2_ref[...] = jnp.maximum(g2_ref[...], m)


def _tc5_body(p_ref, dp_ref, rm_ref, out_ref):
    dinv = 1.0 / (dp_ref[0] + dp_ref[1] + 1e-16)
    dinv48 = jnp.dot(dinv, rm_ref[...], preferred_element_type=jnp.float32)
    o = (p_ref[0] + p_ref[1]) * dinv48
    t = o[:, :NC2]
    m = jnp.max(t, axis=1, keepdims=True)
    lse = jnp.log(jnp.sum(jnp.exp(t - m), axis=1, keepdims=True))
    out_ref[...] = t - m - lse


def _wid_and_niter():
    cid = lax.axis_index("c")
    sid = lax.axis_index("s")
    wid = sid * 2 + cid
    return cid, sid, wid, N_I


def _zero_shared_slice(zbuf, shared, sid):
    pltpu.sync_copy(zbuf.at[pl.ds(0, CHUNK)],
                    shared.at[pl.ds(sid * ROWS_PT, CHUNK)])
    pltpu.sync_copy(zbuf.at[pl.ds(0, ROWS_PT - CHUNK)],
                    shared.at[pl.ds(sid * ROWS_PT + CHUNK, ROWS_PT - CHUNK)])


def _idx16(ref, j):
    return ref[j // (SUB // 16), pl.ds(16 * (j % (SUB // 16)), 16)]


def _sc_a1_body(src_hbm, dst_hbm, as_hbm, ad_hbm, g_hbm,
                eexp_hbm, dpart_hbm,
                srcv0, dstv0, asv0, adv0, eev0,
                srcv1, dstv1, asv1, adv1, eev1,
                gv, dsh, sem0, sem1):
    cid, sid, wid, n_i = _wid_and_niter()
    srcvs = (srcv0, srcv1)
    dstvs = (dstv0, dstv1)
    asvs = (asv0, asv1)
    advs = (adv0, adv1)
    eevs = (eev0, eev1)
    sems = (sem0, sem1)
    zero = jnp.zeros((16,), jnp.float32)

    def zb(j, _):
        eev0[j] = zero
        return 0
    lax.fori_loop(0, CHUNK, zb, 0)
    _zero_shared_slice(eev0, dsh, sid)
    pltpu.sync_copy(g_hbm, gv)
    plsc.subcore_barrier()

    g = gv[...]

    def fire(b, t):
        c = wid + NWORK * t
        i1 = pltpu.async_copy(src_hbm.at[c], srcvs[b], sems[b])
        i2 = pltpu.async_copy(dst_hbm.at[c], dstvs[b], sems[b])
        i1.wait()
        i2.wait()
        for k in range(NSUB):
            pltpu.async_copy(as_hbm.at[srcvs[b].at[k]],
                             asvs[b].at[pl.ds(SUB * k, SUB)], sems[b])
            pltpu.async_copy(ad_hbm.at[dstvs[b].at[k]],
                             advs[b].at[pl.ds(SUB * k, SUB)], sems[b])

    def drain(b):
        for k in range(NSUB):
            pltpu.make_async_copy(as_hbm.at[pl.ds(0, SUB)],
                                  asvs[b].at[pl.ds(SUB * k, SUB)],
                                  sems[b]).wait()
            pltpu.make_async_copy(ad_hbm.at[pl.ds(0, SUB)],
                                  advs[b].at[pl.ds(SUB * k, SUB)],
                                  sems[b]).wait()

    fire(0, 0)

    def pair_body(ip, _):
        for b in (0, 1):
            t = 2 * ip + b
            c = wid + NWORK * t
            drain(b)

            @pl.when(t + 1 < N_I)
            def _():
                fire(1 - b, t + 1)

            def inner(r, _):
                s = asvs[b][r]
                a = advs[b][r]
                e = _leaky(s + a)
                cb = _leaky(g + a)
                eevs[b][r] = jnp.exp(e - cb)
                return 0
            lax.fori_loop(0, CHUNK, inner, 0)

            for k in range(NSUB):
                pltpu.sync_copy(eevs[b].at[pl.ds(SUB * k, SUB)],
                                dsh.at[dstvs[b].at[k]], add=True)
            pltpu.sync_copy(eevs[b], eexp_hbm.at[pl.ds(CHUNK * c, CHUNK)])
        return 0
    lax.fori_loop(0, N_I // 2, pair_body, 0)

    plsc.subcore_barrier()
    pltpu.sync_copy(dsh.at[pl.ds(sid * ROWS_PT, ROWS_PT)],
                    dpart_hbm.at[cid, pl.ds(sid * ROWS_PT, ROWS_PT)])


def _sc_b1_body(src_hbm, dst_hbm, h_hbm, eexp_hbm,
                opart_hbm,
                srcv0, dstv0, hrow0, eev0,
                srcv1, dstv1, hrow1, eev1,
                osh, sem0, sem1):
    cid, sid, wid, n_i = _wid_and_niter()
    srcvs = (srcv0, srcv1)
    dstvs = (dstv0, dstv1)
    hrs = (hrow0, hrow1)
    eevs = (eev0, eev1)
    sems = (sem0, sem1)
    zero = jnp.zeros((16,), jnp.float32)

    def zb(j, _):
        for t in range(4):
            hrow0[j, pl.ds(16 * t, 16)] = zero
        return 0
    lax.fori_loop(0, CHUNK, zb, 0)
    _zero_shared_slice(hrow0, osh, sid)
    plsc.subcore_barrier()

    q = lax.iota(jnp.int32, 16)
    bidx = [2 * t + lax.shift_right_logical(q, 3) for t in range(4)]

    def fire(b, t):
        c = wid + NWORK * t
        i1 = pltpu.async_copy(src_hbm.at[c], srcvs[b], sems[b])
        i2 = pltpu.async_copy(dst_hbm.at[c], dstvs[b], sems[b])
        i1.wait()
        i2.wait()
        pltpu.async_copy(eexp_hbm.at[pl.ds(CHUNK * c, CHUNK)], eevs[b], sems[b])
        for k in range(NSUB):
            pltpu.async_copy(h_hbm.at[srcvs[b].at[k]],
                             hrs[b].at[pl.ds(SUB * k, SUB)], sems[b])

    def drain(b):
        pltpu.make_async_copy(eexp_hbm.at[pl.ds(0, CHUNK)], eevs[b],
                              sems[b]).wait()
        for k in range(NSUB):
            pltpu.make_async_copy(h_hbm.at[pl.ds(0, SUB)],
                                  hrs[b].at[pl.ds(SUB * k, SUB)],
                                  sems[b]).wait()

    fire(0, 0)

    def pair_body(ip, _):
        for b in (0, 1):
            t = 2 * ip + b
            drain(b)

            @pl.when(t + 1 < N_I)
            def _():
                fire(1 - b, t + 1)

            def inner(r, _):
                al = eevs[b][r]
                for tt in range(4):
                    av = _vgather(al, bidx[tt])
                    hrs[b][r, pl.ds(16 * tt, 16)] = (
                        hrs[b][r, pl.ds(16 * tt, 16)] * av)
                return 0
            lax.fori_loop(0, CHUNK, inner, 0)

            for k in range(NSUB):
                pltpu.sync_copy(hrs[b].at[pl.ds(SUB * k, SUB)],
                                osh.at[dstvs[b].at[k]], add=True)
        return 0
    lax.fori_loop(0, N_I // 2, pair_body, 0)

    plsc.subcore_barrier()
    pltpu.sync_copy(osh.at[pl.ds(sid * ROWS_PT, ROWS_PT)],
                    opart_hbm.at[cid, pl.ds(sid * ROWS_PT, ROWS_PT)])


def _sc_a2_body(src_hbm, dst_hbm, as2_hbm, ad2_hbm, g2_hbm,
                eexp2_hbm, d2part_hbm,
                srcv, dstv, as2t, ad2t, eevc, eevw, gv, dsh2, sem):
    cid, sid, wid, n_i = _wid_and_niter()
    zero = jnp.zeros((16,), jnp.float32)

    def zb(j, _):
        eevw[j] = zero
        return 0
    lax.fori_loop(0, CHUNK, zb, 0)
    _zero_shared_slice(eevw, dsh2, sid)
    pltpu.sync_copy(as2_hbm, as2t)
    pltpu.sync_copy(ad2_hbm, ad2t)
    pltpu.sync_copy(g2_hbm, gv)
    plsc.subcore_barrier()

    g = gv[...]
    q = lax.iota(jnp.int32, 16)
    zcol = q * 0

    def chunk_body(i, _):
        c = wid + NWORK * i
        d1 = pltpu.async_copy(src_hbm.at[c], srcv, sem)
        d2 = pltpu.async_copy(dst_hbm.at[c], dstv, sem)
        d1.wait()
        d2.wait()

        def inner(j, _):
            s16 = _idx16(srcv, j)
            d16 = _idx16(dstv, j)
            a_s = plsc.load_gather(as2t, [s16])
            a_d = plsc.load_gather(ad2t, [d16])
            e = _leaky(a_s + a_d)
            cb = _leaky(g + a_d)
            ee = jnp.exp(e - cb)
            eevc[j] = ee
            plsc.store_scatter(eevw, [16 * j + q, zcol], ee)
            return 0
        lax.fori_loop(0, CHUNK // 16, inner, 0)

        for k in range(NSUB):
            pltpu.sync_copy(eevw.at[pl.ds(SUB * k, SUB)], dsh2.at[dstv.at[k]],
                            add=True)
        pltpu.sync_copy(eevc, eexp2_hbm.at[pl.ds(CHUNK // 16 * c, CHUNK // 16)])
        return 0
    lax.fori_loop(0, n_i, chunk_body, 0)

    plsc.subcore_barrier()
    pltpu.sync_copy(dsh2.at[pl.ds(sid * ROWS_PT, ROWS_PT)],
                    d2part_hbm.at[cid, pl.ds(sid * ROWS_PT, ROWS_PT)])


def _sc_b2_body(src_hbm, dst_hbm, h2_hbm, eexp2_hbm,
                o2part_hbm,
                srcv0, dstv0, hrow0, eevc0,
                srcv1, dstv1, hrow1, eevc1,
                osh2, sem0, sem1):
    cid, sid, wid, n_i = _wid_and_niter()
    srcvs = (srcv0, srcv1)
    dstvs = (dstv0, dstv1)
    hrs = (hrow0, hrow1)
    eevcs = (eevc0, eevc1)
    sems = (sem0, sem1)
    zero = jnp.zeros((16,), jnp.float32)

    def zb(j, _):
        for t in range(3):
            hrow0[j, pl.ds(16 * t, 16)] = zero
        return 0
    lax.fori_loop(0, CHUNK, zb, 0)
    _zero_shared_slice(hrow0, osh2, sid)
    plsc.subcore_barrier()

    q = lax.iota(jnp.int32, 16)
    sidx = [q * 0 + k for k in range(16)]

    def fire(b, t):
        c = wid + NWORK * t
        i1 = pltpu.async_copy(src_hbm.at[c], srcvs[b], sems[b])
        i2 = pltpu.async_copy(dst_hbm.at[c], dstvs[b], sems[b])
        i1.wait()
        i2.wait()
        pltpu.async_copy(eexp2_hbm.at[pl.ds(CHUNK // 16 * c, CHUNK // 16)],
                         eevcs[b], sems[b])
        for k in range(NSUB):
            pltpu.async_copy(h2_hbm.at[srcvs[b].at[k]],
                             hrs[b].at[pl.ds(SUB * k, SUB)], sems[b])

    def drain(b):
        pltpu.make_async_copy(eexp2_hbm.at[pl.ds(0, CHUNK // 16)], eevcs[b],
                              sems[b]).wait()
        for k in range(NSUB):
            pltpu.make_async_copy(h2_hbm.at[pl.ds(0, SUB)],
                                  hrs[b].at[pl.ds(SUB * k, SUB)],
                                  sems[b]).wait()

    fire(0, 0)

    def pair_body(ip, _):
        for b in (0, 1):
            t = 2 * ip + b
            drain(b)

            @pl.when(t + 1 < N_I)
            def _():
                fire(1 - b, t + 1)

            def inner(j, _):
                al = eevcs[b][j]
                for k in range(16):
                    av = _vgather(al, sidx[k])
                    r = 16 * j + k
                    for t2 in range(3):
                        hrs[b][r, pl.ds(16 * t2, 16)] = (
                            hrs[b][r, pl.ds(16 * t2, 16)] * av)
                return 0
            lax.fori_loop(0, CHUNK // 16, inner, 0)

            for k in range(NSUB):
                pltpu.sync_copy(hrs[b].at[pl.ds(SUB * k, SUB)],
                                osh2.at[dstvs[b].at[k]], add=True)
        return 0
    lax.fori_loop(0, N_I // 2, pair_body, 0)

    plsc.subcore_barrier()
    pltpu.sync_copy(osh2.at[pl.ds(sid * ROWS_PT, ROWS_PT)],
                    o2part_hbm.at[cid, pl.ds(sid * ROWS_PT, ROWS_PT)])


def kernel(x, edge_index, W1, a1_src, a1_dst, W2, a2_src, a2_dst):
    f32 = jnp.float32
    i32 = jnp.int32
    src = edge_index[0].astype(i32)
    dst = edge_index[1].astype(i32)
    pad_e = NB + jnp.arange(EBP - EB, dtype=i32) % (NBP - NB)
    src2 = jnp.concatenate([src, pad_e]).reshape(NCHUNKS, NSUB, SUB)
    dst2 = jnp.concatenate([dst, pad_e]).reshape(NCHUNKS, NSUB, SUB)

    eye = jnp.eye(NH1, dtype=f32)
    ams = jnp.pad((a1_src[:, :, None] * eye[:, None, :]).reshape(D1, NH1),
                  ((0, 0), (0, 8)))
    amd = jnp.pad((a1_dst[:, :, None] * eye[:, None, :]).reshape(D1, NH1),
                  ((0, 0), (0, 8)))

    h1, as16, ad16, g8 = pl.pallas_call(
        _tc1_body,
        out_shape=(
            jax.ShapeDtypeStruct((NB, D1), f32),
            jax.ShapeDtypeStruct((NB, 16), f32),
            jax.ShapeDtypeStruct((NB, 16), f32),
            jax.ShapeDtypeStruct((1, 16), f32),
        ),
    )(x, W1, ams, amd)
    g16 = jnp.where(jnp.arange(16) < NH1, g8[0], 40.0)
    as16p = jnp.pad(as16, ((0, NBP - NB), (0, 0)))
    ad16p = jnp.pad(ad16, ((0, NBP - NB), (0, 0)))
    h1p = jnp.pad(h1, ((0, NBP - NB), (0, 0)))

    mesh = plsc.VectorSubcoreMesh(core_axis_name="c", subcore_axis_name="s")
    idx_t = pltpu.VMEM((NSUB, SUB), i32)
    sc_params = pltpu.CompilerParams(use_tc_tiling_on_sc=False,
                                     needs_layout_passes=False)

    eexp1, dpart = pl.kernel(
        _sc_a1_body,
        out_type=(
            jax.ShapeDtypeStruct((EBP, 16), f32),
            jax.ShapeDtypeStruct((2, NBP, 16), f32),
        ),
        mesh=mesh,
        compiler_params=sc_params,
        scratch_types=[
            idx_t, idx_t,
            pltpu.VMEM((CHUNK, 16), f32),
            pltpu.VMEM((CHUNK, 16), f32),
            pltpu.VMEM((CHUNK, 16), f32),
            idx_t, idx_t,
            pltpu.VMEM((CHUNK, 16), f32),
            pltpu.VMEM((CHUNK, 16), f32),
            pltpu.VMEM((CHUNK, 16), f32),
            pltpu.VMEM((16,), f32),
            pltpu.VMEM_SHARED((NBP, 16), f32),
            pltpu.SemaphoreType.DMA,
            pltpu.SemaphoreType.DMA,
        ],
    )(src2, dst2, as16p, ad16p, g16)

    opart = pl.kernel(
        _sc_b1_body,
        out_type=jax.ShapeDtypeStruct((2, NBP, D1), f32),
        mesh=mesh,
        compiler_params=sc_params,
        scratch_types=[
            idx_t, idx_t,
            pltpu.VMEM((CHUNK, D1), f32),
            pltpu.VMEM((CHUNK, 16), f32),
            idx_t, idx_t,
            pltpu.VMEM((CHUNK, D1), f32),
            pltpu.VMEM((CHUNK, 16), f32),
            pltpu.VMEM_SHARED((NBP, D1), f32),
            pltpu.SemaphoreType.DMA,
            pltpu.SemaphoreType.DMA,
        ],
    )(src2, dst2, h1p, eexp1)

    w2p = jnp.pad(W2, ((0, 0), (0, D2 - NC2)))
    as2v = jnp.pad(a2_src[0], (0, D2 - NC2)).reshape(D2, 1)
    ad2v = jnp.pad(a2_dst[0], (0, D2 - NC2)).reshape(D2, 1)

    rm1 = (jnp.arange(D1)[None, :] // 8 ==
           jnp.arange(16)[:, None]).astype(f32)
    b3 = 1024
    h2, s2, d2, g2 = pl.pallas_call(
        _tc3_body,
        grid=(NBP // b3,),
        in_specs=[pl.BlockSpec((2, b3, D1), lambda i: (0, i, 0)),
                  pl.BlockSpec((2, b3, 16), lambda i: (0, i, 0)),
                  pl.BlockSpec((16, D1), lambda i: (0, 0)),
                  pl.BlockSpec((D1, D2), lambda i: (0, 0)),
                  pl.BlockSpec((D2, 1), lambda i: (0, 0)),
                  pl.BlockSpec((D2, 1), lambda i: (0, 0))],
        out_specs=(
            pl.BlockSpec((b3, D2), lambda i: (i, 0)),
            pl.BlockSpec((b3, 1), lambda i: (i, 0)),
            pl.BlockSpec((b3, 1), lambda i: (i, 0)),
            pl.BlockSpec((1, 1), lambda i: (0, 0)),
        ),
        out_shape=(
            jax.ShapeDtypeStruct((NBP, D2), f32),
            jax.ShapeDtypeStruct((NBP, 1), f32),
            jax.ShapeDtypeStruct((NBP, 1), f32),
            jax.ShapeDtypeStruct((1, 1), f32),
        ),
    )(opart, dpart, rm1, w2p, as2v, ad2v)
    g2_16 = jnp.broadcast_to(g2[0, 0], (16,))

    eexp2, d2part = pl.kernel(
        _sc_a2_body,
        out_type=(
            jax.ShapeDtypeStruct((EBP // 16, 16), f32),
            jax.ShapeDtypeStruct((2, NBP, 16), f32),
        ),
        mesh=mesh,
        compiler_params=sc_params,
        scratch_types=[
            idx_t, idx_t,
            pltpu.VMEM((NBP,), f32),
            pltpu.VMEM((NBP,), f32),
            pltpu.VMEM((CHUNK // 16, 16), f32),
            pltpu.VMEM((CHUNK, 16), f32),
            pltpu.VMEM((16,), f32),
            pltpu.VMEM_SHARED((NBP, 16), f32),
            pltpu.SemaphoreType.DMA,
        ],
    )(src2, dst2, s2.reshape(NBP), d2.reshape(NBP), g2_16)

    o2part = pl.kernel(
        _sc_b2_body,
        out_type=jax.ShapeDtypeStruct((2, NBP, D2), f32),
        mesh=mesh,
        compiler_params=sc_params,
        scratch_types=[
            idx_t, idx_t,
            pltpu.VMEM((CHUNK, D2), f32),
            pltpu.VMEM((CHUNK // 16, 16), f32),
            idx_t, idx_t,
            pltpu.VMEM((CHUNK, D2), f32),
            pltpu.VMEM((CHUNK // 16, 16), f32),
            pltpu.VMEM_SHARED((NBP, D2), f32),
            pltpu.SemaphoreType.DMA,
            pltpu.SemaphoreType.DMA,
        ],
    )(src2, dst2, h2, eexp2)

    rm2 = (jnp.arange(16)[:, None] * jnp.ones((1, D2)) == 0).astype(f32)
    b5 = 1000
    out = pl.pallas_call(
        _tc5_body,
        grid=(NB // b5,),
        in_specs=[pl.BlockSpec((2, b5, D2), lambda i: (0, i, 0)),
                  pl.BlockSpec((2, b5, 16), lambda i: (0, i, 0)),
                  pl.BlockSpec((16, D2), lambda i: (0, 0))],
        out_specs=pl.BlockSpec((b5, NC2), lambda i: (i, 0)),
        out_shape=jax.ShapeDtypeStruct((NB, NC2), f32),
    )(o2part, d2part, rm2)
    return out
